# Initial kernel scaffold; baseline (speedup 1.0000x reference)
import jax
import jax.numpy as jnp
from jax.experimental import pallas as pl


def kernel(h, x, edge_index, edge_attr, We1, be1, We2, be2, Wh1, bh1, Wh2, bh2, Wx1, bx1, Wx2, bx2, ln_g, ln_b):
    raise NotImplementedError("write your pallas kernel here")


# speedup vs baseline: 3.0219x; 3.0219x over previous
import functools

import jax
import jax.numpy as jnp
from jax import lax
from jax.experimental import pallas as pl
from jax.experimental.pallas import tpu as pltpu
from jax.experimental.pallas import tpu_sc as plsc

_NC = 2
_NS = 16
_NW = _NC * _NS
_CH = 80


def _prep_call(h, A, B):
    N, H = h.shape
    BN = 2000
    def body(h_ref, a_ref, b_ref, p_ref, q_ref):
        hb = h_ref[...]
        p_ref[...] = jnp.dot(hb, a_ref[...], preferred_element_type=jnp.float32)
        q_ref[...] = jnp.dot(hb, b_ref[...], preferred_element_type=jnp.float32)
    return pl.pallas_call(
        body,
        grid=(N // BN,),
        in_specs=[
            pl.BlockSpec((BN, H), lambda i: (i, 0)),
            pl.BlockSpec((H, H), lambda i: (0, 0)),
            pl.BlockSpec((H, H), lambda i: (0, 0)),
        ],
        out_specs=[
            pl.BlockSpec((BN, H), lambda i: (i, 0)),
            pl.BlockSpec((BN, H), lambda i: (i, 0)),
        ],
        out_shape=[
            jax.ShapeDtypeStruct((N, H), jnp.float32),
            jax.ShapeDtypeStruct((N, H), jnp.float32),
        ],
    )(h, A, B)


def _gather_call(P, Q, X16, src2, dst2):
    N, H = P.shape
    ROWS = src2.shape[0]
    RPW = ROWS // _NW
    mesh = plsc.VectorSubcoreMesh(core_axis_name="c", subcore_axis_name="s")

    @functools.partial(
        pl.kernel,
        mesh=mesh,
        compiler_params=pltpu.CompilerParams(use_tc_tiling_on_sc=False),
        out_type=(
            jax.ShapeDtypeStruct((ROWS, _CH, H), jnp.float32),
            jax.ShapeDtypeStruct((ROWS, _CH, 16), jnp.float32),
        ),
        scratch_types=[
            pltpu.VMEM((_CH,), jnp.int32),
            pltpu.VMEM((_CH,), jnp.int32),
            pltpu.VMEM((_CH, H), jnp.float32),
            pltpu.VMEM((_CH, H), jnp.float32),
            pltpu.VMEM((_CH, 16), jnp.float32),
            pltpu.VMEM((_CH, 16), jnp.float32),
            pltpu.SemaphoreType.DMA,
            pltpu.SemaphoreType.DMA,
            pltpu.SemaphoreType.DMA,
            pltpu.SemaphoreType.DMA,
        ],
    )
    def k(p_hbm, q_hbm, x_hbm, src_hbm, dst_hbm, pre_hbm, diff_hbm,
          idx_s, idx_d, buf_p, buf_q, buf_xs, buf_xd, s0, s1, s2, s3):
        wid = lax.axis_index("s") * _NC + lax.axis_index("c")
        base = wid * RPW

        def row_body(i, carry):
            r = base + i
            pltpu.sync_copy(src_hbm.at[r], idx_s)
            pltpu.sync_copy(dst_hbm.at[r], idx_d)
            cp0 = pltpu.async_copy(p_hbm.at[idx_s], buf_p, s0)
            cp1 = pltpu.async_copy(q_hbm.at[idx_d], buf_q, s1)
            cp2 = pltpu.async_copy(x_hbm.at[idx_s], buf_xs, s2)
            cp3 = pltpu.async_copy(x_hbm.at[idx_d], buf_xd, s3)
            cp0.wait()
            cp1.wait()
            cp2.wait()
            cp3.wait()

            def add_body(e, c2):
                for j in range(H // 16):
                    sl = pl.ds(j * 16, 16)
                    buf_p[e, sl] = buf_p[e, sl] + buf_q[e, sl]
                buf_xd[e, :] = buf_xd[e, :] - buf_xs[e, :]
                return c2

            lax.fori_loop(0, _CH, add_body, 0)
            pltpu.sync_copy(buf_p, pre_hbm.at[r])
            pltpu.sync_copy(buf_xd, diff_hbm.at[r])
            return carry

        lax.fori_loop(0, RPW, row_body, 0)

    return k(P, Q, X16, src2, dst2)


def _edge_call(pre_e, diff, edge_attr, C, wd, be1, We2, be2, Wx1, bx1, wx2, bx2):
    E, H = pre_e.shape
    BE = 2000

    def body(pre_ref, diff_ref, ea_ref, c_ref, wd_ref, be1_ref, we2_ref,
             be2_ref, wx1_ref, bx1_ref, wx2_ref, bx2_ref, m_ref, cm_ref):
        d = diff_ref[...]
        sumsq = jnp.sum(d * d, axis=1, keepdims=True)
        dist = jnp.sqrt(sumsq + 1e-9)
        dnorm = jnp.sqrt(sumsq) + 1e-9
        pre = (pre_ref[...]
               + jnp.dot(ea_ref[...], c_ref[...], preferred_element_type=jnp.float32)
               + dist * wd_ref[...] + be1_ref[...])
        m1 = pre * jax.nn.sigmoid(pre)
        m = jnp.dot(m1, we2_ref[...], preferred_element_type=jnp.float32) + be2_ref[...]
        m_ref[...] = m
        g = jnp.dot(m, wx1_ref[...], preferred_element_type=jnp.float32) + bx1_ref[...]
        g = g * jax.nn.sigmoid(g)
        gate = jnp.sum(g * wx2_ref[...], axis=1, keepdims=True) + bx2_ref[...]
        cm_ref[...] = d * (gate / dnorm)

    full = lambda shape: pl.BlockSpec(shape, lambda i: (0,) * len(shape))
    return pl.pallas_call(
        body,
        grid=(E // BE,),
        in_specs=[
            pl.BlockSpec((BE, H), lambda i: (i, 0)),
            pl.BlockSpec((BE, 16), lambda i: (i, 0)),
            pl.BlockSpec((BE, 16), lambda i: (i, 0)),
            full((16, H)), full((1, H)), full((1, H)), full((H, H)),
            full((1, H)), full((H, H)), full((1, H)), full((1, H)),
            full((1, 1)),
        ],
        out_specs=[
            pl.BlockSpec((BE, H), lambda i: (i, 0)),
            pl.BlockSpec((BE, 16), lambda i: (i, 0)),
        ],
        out_shape=[
            jax.ShapeDtypeStruct((E, H), jnp.float32),
            jax.ShapeDtypeStruct((E, 16), jnp.float32),
        ],
    )(pre_e, diff, edge_attr, C, wd, be1, We2, be2, Wx1, bx1, wx2, bx2)


def _scatter_call(m3, cm3, dst2, N):
    ROWS, _, H = m3.shape
    RPW = ROWS // _NW
    RN = N // _NS
    ZR = 125
    mesh = plsc.VectorSubcoreMesh(core_axis_name="c", subcore_axis_name="s")

    @functools.partial(
        pl.kernel,
        mesh=mesh,
        compiler_params=pltpu.CompilerParams(use_tc_tiling_on_sc=False),
        out_type=(
            jax.ShapeDtypeStruct((_NC, N, H), jnp.float32),
            jax.ShapeDtypeStruct((_NC, N, 16), jnp.float32),
        ),
        scratch_types=[
            pltpu.VMEM((_CH,), jnp.int32),
            pltpu.VMEM((_CH, H), jnp.float32),
            pltpu.VMEM((_CH, 16), jnp.float32),
            pltpu.VMEM((ZR, H), jnp.float32),
            pltpu.VMEM((ZR, 16), jnp.float32),
            pltpu.VMEM_SHARED((N, H), jnp.float32),
            pltpu.VMEM_SHARED((N, 16), jnp.float32),
        ],
    )
    def k(m_hbm, cm_hbm, dst_hbm, agg_hbm, dx_hbm,
          idx_d, buf_m, buf_c, z_m, z_c, acc_a, acc_x):
        c = lax.axis_index("c")
        s = lax.axis_index("s")
        wid = s * _NC + c

        def zbody(i, carry):
            for j in range(H // 16):
                z_m[i, pl.ds(j * 16, 16)] = jnp.zeros((16,), jnp.float32)
            z_c[i, :] = jnp.zeros((16,), jnp.float32)
            return carry

        lax.fori_loop(0, ZR, zbody, 0)
        tbase = s * RN
        for kk in range(RN // ZR):
            pltpu.sync_copy(z_m, acc_a.at[pl.ds(tbase + kk * ZR, ZR)])
            pltpu.sync_copy(z_c, acc_x.at[pl.ds(tbase + kk * ZR, ZR)])
        plsc.subcore_barrier()

        base = wid * RPW

        def row_body(i, carry):
            r = base + i
            pltpu.sync_copy(dst_hbm.at[r], idx_d)
            pltpu.sync_copy(m_hbm.at[r], buf_m)
            pltpu.sync_copy(cm_hbm.at[r], buf_c)
            pltpu.sync_copy(buf_m, acc_a.at[idx_d], add=True)
            pltpu.sync_copy(buf_c, acc_x.at[idx_d], add=True)
            return carry

        lax.fori_loop(0, RPW, row_body, 0)
        plsc.subcore_barrier()

        for kk in range(RN // ZR):
            off = tbase + kk * ZR
            pltpu.sync_copy(acc_a.at[pl.ds(off, ZR)], z_m)
            pltpu.sync_copy(z_m, agg_hbm.at[c, pl.ds(off, ZR)])
            pltpu.sync_copy(acc_x.at[pl.ds(off, ZR)], z_c)
            pltpu.sync_copy(z_c, dx_hbm.at[c, pl.ds(off, ZR)])

    return k(m3, cm3, dst2)


def _node_call(h, x16, aggp, dxp, Wh1a, Wh1b, bh1, Wh2, bh2, ln_g, ln_b):
    N, H = h.shape
    BN = 2000

    def body(h_ref, x_ref, agg_ref, dx_ref, wa_ref, wb_ref, bh1_ref,
             wh2_ref, bh2_ref, g_ref, b_ref, ho_ref, xo_ref):
        hb = h_ref[...]
        agg = agg_ref[0] + agg_ref[1]
        t = (jnp.dot(hb, wa_ref[...], preferred_element_type=jnp.float32)
             + jnp.dot(agg, wb_ref[...], preferred_element_type=jnp.float32)
             + bh1_ref[...])
        t = t * jax.nn.sigmoid(t)
        dh = jnp.dot(t, wh2_ref[...], preferred_element_type=jnp.float32) + bh2_ref[...]
        pre = hb + dh
        mu = jnp.mean(pre, axis=1, keepdims=True)
        ctr = pre - mu
        var = jnp.mean(ctr * ctr, axis=1, keepdims=True)
        ho_ref[...] = ctr / jnp.sqrt(var + 1e-5) * g_ref[...] + b_ref[...]
        xo_ref[...] = x_ref[...] + dx_ref[0] + dx_ref[1]

    full = lambda shape: pl.BlockSpec(shape, lambda i: (0,) * len(shape))
    return pl.pallas_call(
        body,
        grid=(N // BN,),
        in_specs=[
            pl.BlockSpec((BN, H), lambda i: (i, 0)),
            pl.BlockSpec((BN, 16), lambda i: (i, 0)),
            pl.BlockSpec((_NC, BN, H), lambda i: (0, i, 0)),
            pl.BlockSpec((_NC, BN, 16), lambda i: (0, i, 0)),
            full((H, H)), full((H, H)), full((1, H)), full((H, H)),
            full((1, H)), full((1, H)), full((1, H)),
        ],
        out_specs=[
            pl.BlockSpec((BN, H), lambda i: (i, 0)),
            pl.BlockSpec((BN, 16), lambda i: (i, 0)),
        ],
        out_shape=[
            jax.ShapeDtypeStruct((N, H), jnp.float32),
            jax.ShapeDtypeStruct((N, 16), jnp.float32),
        ],
    )(h, x16, aggp, dxp, Wh1a, Wh1b, bh1, Wh2, bh2, ln_g, ln_b)


def kernel(h, x, edge_index, edge_attr, We1, be1, We2, be2,
           Wh1, bh1, Wh2, bh2, Wx1, bx1, Wx2, bx2, ln_g, ln_b):
    N, H = h.shape
    E = edge_index.shape[1]
    ED = edge_attr.shape[1]
    ROWS = E // _CH

    src2 = edge_index[0].astype(jnp.int32).reshape(ROWS, _CH)
    dst2 = edge_index[1].astype(jnp.int32).reshape(ROWS, _CH)
    x16 = jnp.pad(x, ((0, 0), (0, 16 - x.shape[1])))

    A = We1[:H]
    B = We1[H:2 * H]
    C = We1[2 * H:2 * H + ED]
    wd = We1[2 * H + ED:2 * H + ED + 1]

    P, Q = _prep_call(h, A, B)
    pre3, diff3 = _gather_call(P, Q, x16, src2, dst2)
    m, cm = _edge_call(
        pre3.reshape(E, H), diff3.reshape(E, 16), edge_attr,
        C, wd, be1.reshape(1, H), We2, be2.reshape(1, H),
        Wx1, bx1.reshape(1, H), Wx2.reshape(1, H), bx2.reshape(1, 1))
    aggp, dxp = _scatter_call(m.reshape(ROWS, _CH, H), cm.reshape(ROWS, _CH, 16), dst2, N)
    h_out, x16o = _node_call(
        h, x16, aggp, dxp, Wh1[:H], Wh1[H:], bh1.reshape(1, H),
        Wh2, bh2.reshape(1, H), ln_g.reshape(1, H), ln_b.reshape(1, H))
    return h_out, x16o[:, :x.shape[1]]


# speedup vs baseline: 4.2858x; 1.4183x over previous
import functools

import jax
import jax.numpy as jnp
from jax import lax
from jax.experimental import pallas as pl
from jax.experimental.pallas import tpu as pltpu
from jax.experimental.pallas import tpu_sc as plsc

_NC = 2
_NS = 16
_NW = _NC * _NS
_CH = 80


def _prep_call(h, A, B):
    N, H = h.shape
    BN = 2000
    def body(h_ref, a_ref, b_ref, p_ref, q_ref):
        hb = h_ref[...]
        p_ref[...] = jnp.dot(hb, a_ref[...], preferred_element_type=jnp.float32)
        q_ref[...] = jnp.dot(hb, b_ref[...], preferred_element_type=jnp.float32)
    return pl.pallas_call(
        body,
        grid=(N // BN,),
        in_specs=[
            pl.BlockSpec((BN, H), lambda i: (i, 0)),
            pl.BlockSpec((H, H), lambda i: (0, 0)),
            pl.BlockSpec((H, H), lambda i: (0, 0)),
        ],
        out_specs=[
            pl.BlockSpec((BN, H), lambda i: (i, 0)),
            pl.BlockSpec((BN, H), lambda i: (i, 0)),
        ],
        out_shape=[
            jax.ShapeDtypeStruct((N, H), jnp.float32),
            jax.ShapeDtypeStruct((N, H), jnp.float32),
        ],
    )(h, A, B)


def _gather_call(P, Q, X16, src2, dst2):
    N, H = P.shape
    ROWS = src2.shape[0]
    RPW = ROWS // _NW
    PAIRS = (RPW - 1) // 2
    mesh = plsc.VectorSubcoreMesh(core_axis_name="c", subcore_axis_name="s")

    vm = pltpu.VMEM
    @functools.partial(
        pl.kernel,
        mesh=mesh,
        compiler_params=pltpu.CompilerParams(use_tc_tiling_on_sc=False),
        out_type=(
            jax.ShapeDtypeStruct((ROWS, _CH, H), jnp.float32),
            jax.ShapeDtypeStruct((ROWS, _CH, 16), jnp.float32),
        ),
        scratch_types=[
            vm((_CH,), jnp.int32), vm((_CH,), jnp.int32),
            vm((_CH,), jnp.int32), vm((_CH,), jnp.int32),
            vm((_CH, H), jnp.float32), vm((_CH, H), jnp.float32),
            vm((_CH, 16), jnp.float32), vm((_CH, 16), jnp.float32),
            vm((_CH, 16), jnp.float32), vm((_CH, 16), jnp.float32),
            pltpu.SemaphoreType.DMA, pltpu.SemaphoreType.DMA,
            pltpu.SemaphoreType.DMA, pltpu.SemaphoreType.DMA,
        ],
    )
    def k(p_hbm, q_hbm, x_hbm, src_hbm, dst_hbm, pre_hbm, diff_hbm,
          is0, id0, is1, id1, p0, p1, xs0, xs1, xd0, xd1,
          sg0, sg1, sx0, sx1):
        wid = lax.axis_index("s") * _NC + lax.axis_index("c")
        base = wid * RPW
        bufs = ((is0, id0, p0, xs0, xd0, sg0, sx0),
                (is1, id1, p1, xs1, xd1, sg1, sx1))

        def start(kk, r):
            is_, id_, p, xs, xd, sg, sx = bufs[kk]
            pltpu.sync_copy(src_hbm.at[r], is_)
            pltpu.sync_copy(dst_hbm.at[r], id_)
            pltpu.async_copy(p_hbm.at[is_], p, sg)
            pltpu.async_copy(x_hbm.at[is_], xs, sx)
            pltpu.async_copy(x_hbm.at[id_], xd, sx)

        def qadd(kk):
            is_, id_, p, xs, xd, sg, sx = bufs[kk]
            pltpu.make_async_copy(p_hbm.at[is_], p, sg).wait()
            pltpu.async_copy(q_hbm.at[id_], p, sg, add=True)

        def finish(kk, r):
            is_, id_, p, xs, xd, sg, sx = bufs[kk]
            pltpu.make_async_copy(x_hbm.at[is_], xs, sx).wait()
            pltpu.make_async_copy(x_hbm.at[id_], xd, sx).wait()

            def diff_body(e, c2):
                xd[e, :] = xd[e, :] - xs[e, :]
                return c2

            lax.fori_loop(0, _CH, diff_body, 0)
            pltpu.make_async_copy(q_hbm.at[id_], p, sg).wait()
            pltpu.sync_copy(p, pre_hbm.at[r])
            pltpu.sync_copy(xd, diff_hbm.at[r])

        start(0, base)

        def body(i, carry):
            a = base + 2 * i
            start(1, a + 1)
            qadd(0)
            finish(0, a)
            start(0, a + 2)
            qadd(1)
            finish(1, a + 1)
            return carry

        lax.fori_loop(0, PAIRS, body, 0)
        qadd(0)
        finish(0, base + RPW - 1)

    return k(P, Q, X16, src2, dst2)


def _edge_call(pre_e, diff, edge_attr, C, wd, be1, We2, be2, Wx1, bx1, wx2, bx2):
    E, H = pre_e.shape
    BE = 2000

    def body(pre_ref, diff_ref, ea_ref, c_ref, wd_ref, be1_ref, we2_ref,
             be2_ref, wx1_ref, bx1_ref, wx2_ref, bx2_ref, m_ref, cm_ref):
        d = diff_ref[...]
        sumsq = jnp.sum(d * d, axis=1, keepdims=True)
        dist = jnp.sqrt(sumsq + 1e-9)
        dnorm = jnp.sqrt(sumsq) + 1e-9
        pre = (pre_ref[...]
               + jnp.dot(ea_ref[...], c_ref[...], preferred_element_type=jnp.float32)
               + dist * wd_ref[...] + be1_ref[...])
        m1 = pre * jax.nn.sigmoid(pre)
        m = jnp.dot(m1, we2_ref[...], preferred_element_type=jnp.float32) + be2_ref[...]
        m_ref[...] = m
        g = jnp.dot(m, wx1_ref[...], preferred_element_type=jnp.float32) + bx1_ref[...]
        g = g * jax.nn.sigmoid(g)
        gate = jnp.sum(g * wx2_ref[...], axis=1, keepdims=True) + bx2_ref[...]
        cm_ref[...] = d * (gate / dnorm)

    full = lambda shape: pl.BlockSpec(shape, lambda i: (0,) * len(shape))
    return pl.pallas_call(
        body,
        grid=(E // BE,),
        in_specs=[
            pl.BlockSpec((BE, H), lambda i: (i, 0)),
            pl.BlockSpec((BE, 16), lambda i: (i, 0)),
            pl.BlockSpec((BE, 16), lambda i: (i, 0)),
            full((16, H)), full((1, H)), full((1, H)), full((H, H)),
            full((1, H)), full((H, H)), full((1, H)), full((1, H)),
            full((1, 1)),
        ],
        out_specs=[
            pl.BlockSpec((BE, H), lambda i: (i, 0)),
            pl.BlockSpec((BE, 16), lambda i: (i, 0)),
        ],
        out_shape=[
            jax.ShapeDtypeStruct((E, H), jnp.float32),
            jax.ShapeDtypeStruct((E, 16), jnp.float32),
        ],
    )(pre_e, diff, edge_attr, C, wd, be1, We2, be2, Wx1, bx1, wx2, bx2)


_CHS = 40


def _scatter_call(m3, cm3, dst2, N):
    ROWS, _, H = m3.shape
    RPW = ROWS // _NW
    TAIL = RPW - 4 * (RPW // 4)
    RN = N // _NS
    ZR = 25
    mesh = plsc.VectorSubcoreMesh(core_axis_name="c", subcore_axis_name="s")

    @functools.partial(
        pl.kernel,
        mesh=mesh,
        compiler_params=pltpu.CompilerParams(use_tc_tiling_on_sc=False),
        out_type=(
            jax.ShapeDtypeStruct((_NC, N, H), jnp.float32),
            jax.ShapeDtypeStruct((_NC, N, 16), jnp.float32),
        ),
        scratch_types=[
            pltpu.VMEM((_CHS,), jnp.int32), pltpu.VMEM((_CHS,), jnp.int32),
            pltpu.VMEM((_CHS,), jnp.int32), pltpu.VMEM((_CHS,), jnp.int32),
            pltpu.VMEM((_CHS, H), jnp.float32), pltpu.VMEM((_CHS, H), jnp.float32),
            pltpu.VMEM((_CHS, H), jnp.float32), pltpu.VMEM((_CHS, H), jnp.float32),
            pltpu.VMEM((_CHS, 16), jnp.float32), pltpu.VMEM((_CHS, 16), jnp.float32),
            pltpu.VMEM((_CHS, 16), jnp.float32), pltpu.VMEM((_CHS, 16), jnp.float32),
            pltpu.VMEM((ZR, H), jnp.float32),
            pltpu.VMEM((ZR, 16), jnp.float32),
            pltpu.VMEM_SHARED((N, H), jnp.float32),
            pltpu.VMEM_SHARED((N, 16), jnp.float32),
            pltpu.SemaphoreType.DMA, pltpu.SemaphoreType.DMA,
            pltpu.SemaphoreType.DMA, pltpu.SemaphoreType.DMA,
            pltpu.SemaphoreType.DMA, pltpu.SemaphoreType.DMA,
            pltpu.SemaphoreType.DMA, pltpu.SemaphoreType.DMA,
        ],
    )
    def k(m_hbm, cm_hbm, dst_hbm, agg_hbm, dx_hbm,
          ix0, ix1, ix2, ix3, bm0, bm1, bm2, bm3, bc0, bc1, bc2, bc3,
          z_m, z_c, acc_a, acc_x,
          sl0, sl1, sl2, sl3, ss0, ss1, ss2, ss3):
        c = lax.axis_index("c")
        s = lax.axis_index("s")
        wid = s * _NC + c
        bufs = ((ix0, bm0, bc0, sl0, ss0), (ix1, bm1, bc1, sl1, ss1),
                (ix2, bm2, bc2, sl2, ss2), (ix3, bm3, bc3, sl3, ss3))

        def zbody(i, carry):
            for j in range(H // 16):
                z_m[i, pl.ds(j * 16, 16)] = jnp.zeros((16,), jnp.float32)
            z_c[i, :] = jnp.zeros((16,), jnp.float32)
            return carry

        lax.fori_loop(0, ZR, zbody, 0)
        tbase = s * RN
        for kk in range(RN // ZR):
            pltpu.sync_copy(z_m, acc_a.at[pl.ds(tbase + kk * ZR, ZR)])
            pltpu.sync_copy(z_c, acc_x.at[pl.ds(tbase + kk * ZR, ZR)])
        plsc.subcore_barrier()

        base = wid * RPW

        def load(kk, r):
            ix, bm, bc, sl, ss = bufs[kk]
            pltpu.sync_copy(dst_hbm.at[r], ix)
            pltpu.async_copy(m_hbm.at[r], bm, sl)
            pltpu.async_copy(cm_hbm.at[r], bc, sl)

        def scatter(kk, r):
            ix, bm, bc, sl, ss = bufs[kk]
            pltpu.make_async_copy(m_hbm.at[r], bm, sl).wait()
            pltpu.make_async_copy(cm_hbm.at[r], bc, sl).wait()
            pltpu.async_copy(bm, acc_a.at[ix], ss, add=True)
            pltpu.async_copy(bc, acc_x.at[ix], ss, add=True)

        def wait_scatter(kk):
            ix, bm, bc, sl, ss = bufs[kk]
            pltpu.make_async_copy(bm, acc_a.at[ix], ss).wait()
            pltpu.make_async_copy(bc, acc_x.at[ix], ss).wait()

        for kk in range(4):
            load(kk, base + kk)

        def row_body(i, carry):
            r4 = base + 4 * i
            for kk in range(4):
                scatter(kk, r4 + kk)
            for kk in range(4):
                wait_scatter(kk)

                @pl.when(4 * i + kk + 4 < RPW)
                def _():
                    load(kk, r4 + kk + 4)
            return carry

        lax.fori_loop(0, RPW // 4, row_body, 0)
        for kk in range(TAIL):
            scatter(kk, base + RPW - TAIL + kk)
        for kk in range(TAIL):
            wait_scatter(kk)
        plsc.subcore_barrier()

        for kk in range(RN // ZR):
            off = tbase + kk * ZR
            pltpu.sync_copy(acc_a.at[pl.ds(off, ZR)], z_m)
            pltpu.sync_copy(z_m, agg_hbm.at[c, pl.ds(off, ZR)])
            pltpu.sync_copy(acc_x.at[pl.ds(off, ZR)], z_c)
            pltpu.sync_copy(z_c, dx_hbm.at[c, pl.ds(off, ZR)])

    return k(m3, cm3, dst2)


def _node_call(h, x16, aggp, dxp, Wh1a, Wh1b, bh1, Wh2, bh2, ln_g, ln_b):
    N, H = h.shape
    BN = 2000

    def body(h_ref, x_ref, agg_ref, dx_ref, wa_ref, wb_ref, bh1_ref,
             wh2_ref, bh2_ref, g_ref, b_ref, ho_ref, xo_ref):
        hb = h_ref[...]
        agg = agg_ref[0] + agg_ref[1]
        t = (jnp.dot(hb, wa_ref[...], preferred_element_type=jnp.float32)
             + jnp.dot(agg, wb_ref[...], preferred_element_type=jnp.float32)
             + bh1_ref[...])
        t = t * jax.nn.sigmoid(t)
        dh = jnp.dot(t, wh2_ref[...], preferred_element_type=jnp.float32) + bh2_ref[...]
        pre = hb + dh
        mu = jnp.mean(pre, axis=1, keepdims=True)
        ctr = pre - mu
        var = jnp.mean(ctr * ctr, axis=1, keepdims=True)
        ho_ref[...] = ctr / jnp.sqrt(var + 1e-5) * g_ref[...] + b_ref[...]
        xo_ref[...] = x_ref[...] + dx_ref[0] + dx_ref[1]

    full = lambda shape: pl.BlockSpec(shape, lambda i: (0,) * len(shape))
    return pl.pallas_call(
        body,
        grid=(N // BN,),
        in_specs=[
            pl.BlockSpec((BN, H), lambda i: (i, 0)),
            pl.BlockSpec((BN, 16), lambda i: (i, 0)),
            pl.BlockSpec((_NC, BN, H), lambda i: (0, i, 0)),
            pl.BlockSpec((_NC, BN, 16), lambda i: (0, i, 0)),
            full((H, H)), full((H, H)), full((1, H)), full((H, H)),
            full((1, H)), full((1, H)), full((1, H)),
        ],
        out_specs=[
            pl.BlockSpec((BN, H), lambda i: (i, 0)),
            pl.BlockSpec((BN, 16), lambda i: (i, 0)),
        ],
        out_shape=[
            jax.ShapeDtypeStruct((N, H), jnp.float32),
            jax.ShapeDtypeStruct((N, 16), jnp.float32),
        ],
    )(h, x16, aggp, dxp, Wh1a, Wh1b, bh1, Wh2, bh2, ln_g, ln_b)


def kernel(h, x, edge_index, edge_attr, We1, be1, We2, be2,
           Wh1, bh1, Wh2, bh2, Wx1, bx1, Wx2, bx2, ln_g, ln_b):
    N, H = h.shape
    E = edge_index.shape[1]
    ED = edge_attr.shape[1]
    ROWS = E // _CH

    src2 = edge_index[0].astype(jnp.int32).reshape(ROWS, _CH)
    dst2 = edge_index[1].astype(jnp.int32).reshape(ROWS, _CH)
    x16 = jnp.pad(x, ((0, 0), (0, 16 - x.shape[1])))

    A = We1[:H]
    B = We1[H:2 * H]
    C = We1[2 * H:2 * H + ED]
    wd = We1[2 * H + ED:2 * H + ED + 1]

    P, Q = _prep_call(h, A, B)
    pre3, diff3 = _gather_call(P, Q, x16, src2, dst2)
    m, cm = _edge_call(
        pre3.reshape(E, H), diff3.reshape(E, 16), edge_attr,
        C, wd, be1.reshape(1, H), We2, be2.reshape(1, H),
        Wx1, bx1.reshape(1, H), Wx2.reshape(1, H), bx2.reshape(1, 1))
    ROWS_S = E // _CHS
    dst2s = edge_index[1].astype(jnp.int32).reshape(ROWS_S, _CHS)
    aggp, dxp = _scatter_call(
        m.reshape(ROWS_S, _CHS, H), cm.reshape(ROWS_S, _CHS, 16), dst2s, N)
    h_out, x16o = _node_call(
        h, x16, aggp, dxp, Wh1[:H], Wh1[H:], bh1.reshape(1, H),
        Wh2, bh2.reshape(1, H), ln_g.reshape(1, H), ln_b.reshape(1, H))
    return h_out, x16o[:, :x.shape[1]]


# speedup vs baseline: 4.8741x; 1.1372x over previous
import functools

import jax
import jax.numpy as jnp
from jax import lax
from jax.experimental import pallas as pl
from jax.experimental.pallas import tpu as pltpu
from jax.experimental.pallas import tpu_sc as plsc

def _sigmoid(v):
    return 1.0 / (1.0 + jnp.exp2(v * -1.4426950408889634))


_NC = 2
_NS = 16
_NW = _NC * _NS
_CH = 80


def _prep_call(h, A, B):
    N, H = h.shape
    BN = 2000
    def body(h_ref, a_ref, b_ref, p_ref, q_ref):
        hb = h_ref[...]
        p_ref[...] = jnp.dot(hb, a_ref[...], preferred_element_type=jnp.float32)
        q_ref[...] = jnp.dot(hb, b_ref[...], preferred_element_type=jnp.float32)
    return pl.pallas_call(
        body,
        grid=(N // BN,),
        in_specs=[
            pl.BlockSpec((BN, H), lambda i: (i, 0)),
            pl.BlockSpec((H, H), lambda i: (0, 0)),
            pl.BlockSpec((H, H), lambda i: (0, 0)),
        ],
        out_specs=[
            pl.BlockSpec((BN, H), lambda i: (i, 0)),
            pl.BlockSpec((BN, H), lambda i: (i, 0)),
        ],
        out_shape=[
            jax.ShapeDtypeStruct((N, H), jnp.float32),
            jax.ShapeDtypeStruct((N, H), jnp.float32),
        ],
    )(h, A, B)


def _gather_call(P, Q, X16, src2, dst2):
    N, H = P.shape
    ROWS = src2.shape[0]
    RPW = ROWS // _NW
    PAIRS = (RPW - 1) // 2
    mesh = plsc.VectorSubcoreMesh(core_axis_name="c", subcore_axis_name="s")

    vm = pltpu.VMEM
    @functools.partial(
        pl.kernel,
        mesh=mesh,
        compiler_params=pltpu.CompilerParams(use_tc_tiling_on_sc=False),
        out_type=(
            jax.ShapeDtypeStruct((ROWS, _CH, H), jnp.float32),
            jax.ShapeDtypeStruct((ROWS, _CH, 16), jnp.float32),
        ),
        scratch_types=[
            vm((_CH,), jnp.int32), vm((_CH,), jnp.int32),
            vm((_CH,), jnp.int32), vm((_CH,), jnp.int32),
            vm((_CH, H), jnp.float32), vm((_CH, H), jnp.float32),
            vm((_CH, 16), jnp.float32), vm((_CH, 16), jnp.float32),
            vm((_CH, 16), jnp.float32), vm((_CH, 16), jnp.float32),
            pltpu.SemaphoreType.DMA, pltpu.SemaphoreType.DMA,
            pltpu.SemaphoreType.DMA, pltpu.SemaphoreType.DMA,
        ],
    )
    def k(p_hbm, q_hbm, x_hbm, src_hbm, dst_hbm, pre_hbm, diff_hbm,
          is0, id0, is1, id1, p0, p1, xs0, xs1, xd0, xd1,
          sg0, sg1, sx0, sx1):
        wid = lax.axis_index("s") * _NC + lax.axis_index("c")
        base = wid * RPW
        bufs = ((is0, id0, p0, xs0, xd0, sg0, sx0),
                (is1, id1, p1, xs1, xd1, sg1, sx1))

        def start(kk, r):
            is_, id_, p, xs, xd, sg, sx = bufs[kk]
            pltpu.sync_copy(src_hbm.at[r], is_)
            pltpu.sync_copy(dst_hbm.at[r], id_)
            pltpu.async_copy(p_hbm.at[is_], p, sg)
            pltpu.async_copy(x_hbm.at[is_], xs, sx)
            pltpu.async_copy(x_hbm.at[id_], xd, sx)

        def qadd(kk):
            is_, id_, p, xs, xd, sg, sx = bufs[kk]
            pltpu.make_async_copy(p_hbm.at[is_], p, sg).wait()
            pltpu.async_copy(q_hbm.at[id_], p, sg, add=True)

        def finish(kk, r):
            is_, id_, p, xs, xd, sg, sx = bufs[kk]
            pltpu.make_async_copy(x_hbm.at[is_], xs, sx).wait()
            pltpu.make_async_copy(x_hbm.at[id_], xd, sx).wait()

            def diff_body(e, c2):
                xd[e, :] = xd[e, :] - xs[e, :]
                return c2

            lax.fori_loop(0, _CH, diff_body, 0)
            pltpu.make_async_copy(q_hbm.at[id_], p, sg).wait()
            pltpu.sync_copy(p, pre_hbm.at[r])
            pltpu.sync_copy(xd, diff_hbm.at[r])

        start(0, base)

        def body(i, carry):
            a = base + 2 * i
            start(1, a + 1)
            qadd(0)
            finish(0, a)
            start(0, a + 2)
            qadd(1)
            finish(1, a + 1)
            return carry

        lax.fori_loop(0, PAIRS, body, 0)
        qadd(0)
        finish(0, base + RPW - 1)

    return k(P, Q, X16, src2, dst2)


def _edge_call(pre_e, diff, edge_attr, C, wd, be1, We2, be2, Wx1, bx1, wx2, bx2):
    E, H = pre_e.shape
    BE = 2000

    def body(pre_ref, diff_ref, ea_ref, c_ref, wd_ref, be1_ref, we2_ref,
             be2_ref, wx1_ref, bx1_ref, wx2_ref, bx2_ref, m_ref, cm_ref):
        d = diff_ref[...]
        dsq = d * d
        ones_col = jnp.ones((dsq.shape[1], 1), jnp.float32)
        s = jnp.dot(dsq, ones_col, preferred_element_type=jnp.float32) + 1e-9
        r = lax.rsqrt(s)
        dist = s * r
        pre = (pre_ref[...]
               + jnp.dot(ea_ref[...], c_ref[...], preferred_element_type=jnp.float32)
               + dist * wd_ref[...] + be1_ref[...])
        m1 = pre * _sigmoid(pre)
        m = jnp.dot(m1, we2_ref[...], preferred_element_type=jnp.float32) + be2_ref[...]
        m_ref[...] = m
        g = jnp.dot(m, wx1_ref[...], preferred_element_type=jnp.float32) + bx1_ref[...]
        g = g * _sigmoid(g)
        gate = jnp.dot(g, wx2_ref[...], preferred_element_type=jnp.float32) + bx2_ref[...]
        cm_ref[...] = d * (gate * r)

    full = lambda shape: pl.BlockSpec(shape, lambda i: (0,) * len(shape))
    return pl.pallas_call(
        body,
        grid=(E // BE,),
        in_specs=[
            pl.BlockSpec((BE, H), lambda i: (i, 0)),
            pl.BlockSpec((BE, 16), lambda i: (i, 0)),
            pl.BlockSpec((BE, 16), lambda i: (i, 0)),
            full((16, H)), full((1, H)), full((1, H)), full((H, H)),
            full((1, H)), full((H, H)), full((1, H)), full((H, 1)),
            full((1, 1)),
        ],
        out_specs=[
            pl.BlockSpec((BE, H), lambda i: (i, 0)),
            pl.BlockSpec((BE, 16), lambda i: (i, 0)),
        ],
        out_shape=[
            jax.ShapeDtypeStruct((E, H), jnp.float32),
            jax.ShapeDtypeStruct((E, 16), jnp.float32),
        ],
    )(pre_e, diff, edge_attr, C, wd, be1, We2, be2, Wx1, bx1, wx2, bx2)


_CHS = 40


def _scatter_call(m3, cm3, dst2, N):
    ROWS, _, H = m3.shape
    RPW = ROWS // _NW
    TAIL = RPW - 4 * (RPW // 4)
    RN = N // _NS
    ZR = 25
    mesh = plsc.VectorSubcoreMesh(core_axis_name="c", subcore_axis_name="s")

    @functools.partial(
        pl.kernel,
        mesh=mesh,
        compiler_params=pltpu.CompilerParams(use_tc_tiling_on_sc=False),
        out_type=(
            jax.ShapeDtypeStruct((_NC, N, H), jnp.float32),
            jax.ShapeDtypeStruct((_NC, N, 16), jnp.float32),
        ),
        scratch_types=[
            pltpu.VMEM((_CHS,), jnp.int32), pltpu.VMEM((_CHS,), jnp.int32),
            pltpu.VMEM((_CHS,), jnp.int32), pltpu.VMEM((_CHS,), jnp.int32),
            pltpu.VMEM((_CHS, H), jnp.float32), pltpu.VMEM((_CHS, H), jnp.float32),
            pltpu.VMEM((_CHS, H), jnp.float32), pltpu.VMEM((_CHS, H), jnp.float32),
            pltpu.VMEM((_CHS, 16), jnp.float32), pltpu.VMEM((_CHS, 16), jnp.float32),
            pltpu.VMEM((_CHS, 16), jnp.float32), pltpu.VMEM((_CHS, 16), jnp.float32),
            pltpu.VMEM((ZR, H), jnp.float32),
            pltpu.VMEM((ZR, 16), jnp.float32),
            pltpu.VMEM_SHARED((N, H), jnp.float32),
            pltpu.VMEM_SHARED((N, 16), jnp.float32),
            pltpu.SemaphoreType.DMA, pltpu.SemaphoreType.DMA,
            pltpu.SemaphoreType.DMA, pltpu.SemaphoreType.DMA,
            pltpu.SemaphoreType.DMA, pltpu.SemaphoreType.DMA,
            pltpu.SemaphoreType.DMA, pltpu.SemaphoreType.DMA,
        ],
    )
    def k(m_hbm, cm_hbm, dst_hbm, agg_hbm, dx_hbm,
          ix0, ix1, ix2, ix3, bm0, bm1, bm2, bm3, bc0, bc1, bc2, bc3,
          z_m, z_c, acc_a, acc_x,
          sl0, sl1, sl2, sl3, ss0, ss1, ss2, ss3):
        c = lax.axis_index("c")
        s = lax.axis_index("s")
        wid = s * _NC + c
        bufs = ((ix0, bm0, bc0, sl0, ss0), (ix1, bm1, bc1, sl1, ss1),
                (ix2, bm2, bc2, sl2, ss2), (ix3, bm3, bc3, sl3, ss3))

        def zbody(i, carry):
            for j in range(H // 16):
                z_m[i, pl.ds(j * 16, 16)] = jnp.zeros((16,), jnp.float32)
            z_c[i, :] = jnp.zeros((16,), jnp.float32)
            return carry

        lax.fori_loop(0, ZR, zbody, 0)
        tbase = s * RN
        for kk in range(RN // ZR):
            pltpu.sync_copy(z_m, acc_a.at[pl.ds(tbase + kk * ZR, ZR)])
            pltpu.sync_copy(z_c, acc_x.at[pl.ds(tbase + kk * ZR, ZR)])
        plsc.subcore_barrier()

        base = wid * RPW

        def load(kk, r):
            ix, bm, bc, sl, ss = bufs[kk]
            pltpu.sync_copy(dst_hbm.at[r], ix)
            pltpu.async_copy(m_hbm.at[r], bm, sl)
            pltpu.async_copy(cm_hbm.at[r], bc, sl)

        def scatter(kk, r):
            ix, bm, bc, sl, ss = bufs[kk]
            pltpu.make_async_copy(m_hbm.at[r], bm, sl).wait()
            pltpu.make_async_copy(cm_hbm.at[r], bc, sl).wait()
            pltpu.async_copy(bm, acc_a.at[ix], ss, add=True)
            pltpu.async_copy(bc, acc_x.at[ix], ss, add=True)

        def wait_scatter(kk):
            ix, bm, bc, sl, ss = bufs[kk]
            pltpu.make_async_copy(bm, acc_a.at[ix], ss).wait()
            pltpu.make_async_copy(bc, acc_x.at[ix], ss).wait()

        for kk in range(4):
            load(kk, base + kk)

        def row_body(i, carry):
            r4 = base + 4 * i
            for kk in range(4):
                scatter(kk, r4 + kk)
            for kk in range(4):
                wait_scatter(kk)

                @pl.when(4 * i + kk + 4 < RPW)
                def _():
                    load(kk, r4 + kk + 4)
            return carry

        lax.fori_loop(0, RPW // 4, row_body, 0)
        for kk in range(TAIL):
            scatter(kk, base + RPW - TAIL + kk)
        for kk in range(TAIL):
            wait_scatter(kk)
        plsc.subcore_barrier()

        for kk in range(RN // ZR):
            off = tbase + kk * ZR
            pltpu.sync_copy(acc_a.at[pl.ds(off, ZR)], z_m)
            pltpu.sync_copy(z_m, agg_hbm.at[c, pl.ds(off, ZR)])
            pltpu.sync_copy(acc_x.at[pl.ds(off, ZR)], z_c)
            pltpu.sync_copy(z_c, dx_hbm.at[c, pl.ds(off, ZR)])

    return k(m3, cm3, dst2)


def _node_call(h, x16, aggp, dxp, Wh1a, Wh1b, bh1, Wh2, bh2, ln_g, ln_b):
    N, H = h.shape
    BN = 2000

    def body(h_ref, x_ref, agg_ref, dx_ref, wa_ref, wb_ref, bh1_ref,
             wh2_ref, bh2_ref, g_ref, b_ref, ho_ref, xo_ref):
        hb = h_ref[...]
        agg = agg_ref[0] + agg_ref[1]
        t = (jnp.dot(hb, wa_ref[...], preferred_element_type=jnp.float32)
             + jnp.dot(agg, wb_ref[...], preferred_element_type=jnp.float32)
             + bh1_ref[...])
        t = t * _sigmoid(t)
        dh = jnp.dot(t, wh2_ref[...], preferred_element_type=jnp.float32) + bh2_ref[...]
        pre = hb + dh
        mu = jnp.mean(pre, axis=1, keepdims=True)
        ctr = pre - mu
        var = jnp.mean(ctr * ctr, axis=1, keepdims=True)
        ho_ref[...] = ctr / jnp.sqrt(var + 1e-5) * g_ref[...] + b_ref[...]
        xo_ref[...] = x_ref[...] + dx_ref[0] + dx_ref[1]

    full = lambda shape: pl.BlockSpec(shape, lambda i: (0,) * len(shape))
    return pl.pallas_call(
        body,
        grid=(N // BN,),
        in_specs=[
            pl.BlockSpec((BN, H), lambda i: (i, 0)),
            pl.BlockSpec((BN, 16), lambda i: (i, 0)),
            pl.BlockSpec((_NC, BN, H), lambda i: (0, i, 0)),
            pl.BlockSpec((_NC, BN, 16), lambda i: (0, i, 0)),
            full((H, H)), full((H, H)), full((1, H)), full((H, H)),
            full((1, H)), full((1, H)), full((1, H)),
        ],
        out_specs=[
            pl.BlockSpec((BN, H), lambda i: (i, 0)),
            pl.BlockSpec((BN, 16), lambda i: (i, 0)),
        ],
        out_shape=[
            jax.ShapeDtypeStruct((N, H), jnp.float32),
            jax.ShapeDtypeStruct((N, 16), jnp.float32),
        ],
    )(h, x16, aggp, dxp, Wh1a, Wh1b, bh1, Wh2, bh2, ln_g, ln_b)


def kernel(h, x, edge_index, edge_attr, We1, be1, We2, be2,
           Wh1, bh1, Wh2, bh2, Wx1, bx1, Wx2, bx2, ln_g, ln_b):
    N, H = h.shape
    E = edge_index.shape[1]
    ED = edge_attr.shape[1]
    ROWS = E // _CH

    src2 = edge_index[0].astype(jnp.int32).reshape(ROWS, _CH)
    dst2 = edge_index[1].astype(jnp.int32).reshape(ROWS, _CH)
    x16 = jnp.pad(x, ((0, 0), (0, 16 - x.shape[1])))

    A = We1[:H]
    B = We1[H:2 * H]
    C = We1[2 * H:2 * H + ED]
    wd = We1[2 * H + ED:2 * H + ED + 1]

    P, Q = _prep_call(h, A, B)
    pre3, diff3 = _gather_call(P, Q, x16, src2, dst2)
    m, cm = _edge_call(
        pre3.reshape(E, H), diff3.reshape(E, 16), edge_attr,
        C, wd, be1.reshape(1, H), We2, be2.reshape(1, H),
        Wx1, bx1.reshape(1, H), Wx2, bx2.reshape(1, 1))
    ROWS_S = E // _CHS
    dst2s = edge_index[1].astype(jnp.int32).reshape(ROWS_S, _CHS)
    aggp, dxp = _scatter_call(
        m.reshape(ROWS_S, _CHS, H), cm.reshape(ROWS_S, _CHS, 16), dst2s, N)
    h_out, x16o = _node_call(
        h, x16, aggp, dxp, Wh1[:H], Wh1[H:], bh1.reshape(1, H),
        Wh2, bh2.reshape(1, H), ln_g.reshape(1, H), ln_b.reshape(1, H))
    return h_out, x16o[:, :x.shape[1]]
